# baseline (speedup 1.0000x reference)
import functools

import jax
import jax.numpy as jnp
from jax import lax
from jax.experimental import pallas as pl
from jax.experimental.pallas import tpu as pltpu
from jax.experimental.pallas import tpu_sc as plsc

_B = 16384
_D = 512
_DW = _D // 2
_NUM_CLASSES = 1000
_LAMBDA = 0.001

_NC, _NS, _L = 2, 16, 16
_NW = _NC * _NS
_BPW = _B // _NW
_CH = 64
_NCHUNK = _BPW // _CH

_mesh = plsc.VectorSubcoreMesh(
    core_axis_name="c", subcore_axis_name="s",
    num_cores=_NC, num_subcores=_NS,
)


@functools.partial(
    pl.kernel,
    out_type=jax.ShapeDtypeStruct((_NW, _L), jnp.float32),
    mesh=_mesh,
    scratch_types=[
        pltpu.VMEM((_BPW,), jnp.int32),
        pltpu.VMEM((2, _CH, _DW), jnp.int32),
        pltpu.VMEM((2, _CH, _DW), jnp.int32),
        pltpu.VMEM((_L,), jnp.float32),
        pltpu.SemaphoreType.DMA,
        pltpu.SemaphoreType.DMA,
    ],
)
def _center_loss_sc(emb_hbm, lab_hbm, cen_hbm, out_hbm,
                    idx_v, e_v, c_v, acc_v, sem0, sem1):
    wid = lax.axis_index("s") * _NC + lax.axis_index("c")
    base = wid * _BPW
    pltpu.sync_copy(lab_hbm.at[pl.ds(base, _BPW)], idx_v)
    sems = (sem0, sem1)

    def start(ch, b):
        pltpu.async_copy(
            cen_hbm.at[idx_v.at[pl.ds(ch * _CH, _CH)]], c_v.at[b], sems[b])
        pltpu.async_copy(
            emb_hbm.at[pl.ds(base + ch * _CH, _CH), :], e_v.at[b], sems[b])

    def wait(b):
        pltpu.make_async_copy(emb_hbm.at[pl.ds(0, _CH), :], e_v.at[b],
                              sems[b]).wait()
        pltpu.make_async_copy(emb_hbm.at[pl.ds(0, _CH), :], c_v.at[b],
                              sems[b]).wait()

    hi_mask = jnp.full((_L,), jnp.int32(-65536))

    def expand(w):
        lo = lax.bitcast_convert_type(lax.shift_left(w, 16), jnp.float32)
        hi = lax.bitcast_convert_type(lax.bitwise_and(w, hi_mask), jnp.float32)
        return lo, hi

    def compute(b, a):
        def body(r, a):
            a0, a1, a2, a3 = a
            for j in range(_DW // (2 * _L)):
                ew0 = e_v[b, r, pl.ds((2 * j) * _L, _L)]
                cw0 = c_v[b, r, pl.ds((2 * j) * _L, _L)]
                ew1 = e_v[b, r, pl.ds((2 * j + 1) * _L, _L)]
                cw1 = c_v[b, r, pl.ds((2 * j + 1) * _L, _L)]
                elo0, ehi0 = expand(ew0)
                clo0, chi0 = expand(cw0)
                elo1, ehi1 = expand(ew1)
                clo1, chi1 = expand(cw1)
                d0 = elo0 - clo0
                d1 = ehi0 - chi0
                d2 = elo1 - clo1
                d3 = ehi1 - chi1
                a0 = a0 + d0 * d0
                a1 = a1 + d1 * d1
                a2 = a2 + d2 * d2
                a3 = a3 + d3 * d3
            return a0, a1, a2, a3
        return lax.fori_loop(0, _CH, body, a)

    start(0, 0)
    npairs = _NCHUNK // 2
    zero = jnp.zeros((_L,), jnp.float32)

    def pair_body(p, a):
        start(2 * p + 1, 1)
        wait(0)
        a = compute(0, a)

        @pl.when(p + 1 < npairs)
        def _():
            start(2 * p + 2, 0)

        wait(1)
        a = compute(1, a)
        return a

    a0, a1, a2, a3 = lax.fori_loop(0, npairs, pair_body,
                                   (zero, zero, zero, zero))

    acc_v[...] = (a0 + a1) + (a2 + a3)
    pltpu.sync_copy(acc_v, out_hbm.at[wid])


def _pack_bf16(x):
    b = x.astype(jnp.bfloat16).reshape(x.shape[0], x.shape[1] // 2, 2)
    return jax.lax.bitcast_convert_type(b, jnp.int32)


def kernel(embeddings, labels, centers):
    partials = _center_loss_sc(
        _pack_bf16(embeddings),
        labels.astype(jnp.int32),
        _pack_bf16(centers),
    )
    return _LAMBDA * (jnp.sum(partials) / jnp.float32(_B))


# speedup vs baseline: 2.5389x; 2.5389x over previous
import functools

import jax
import jax.numpy as jnp
from jax import lax
from jax.experimental import pallas as pl
from jax.experimental.pallas import tpu as pltpu
from jax.experimental.pallas import tpu_sc as plsc

_B = 16384
_D = 512
_DW = _D // 2
_NUM_CLASSES = 1000
_LAMBDA = 0.001

_NC, _NS, _L = 2, 16, 16
_NW = _NC * _NS
_BPW = _B // _NW
_CH = 64
_NCHUNK = _BPW // _CH

_mesh = plsc.VectorSubcoreMesh(
    core_axis_name="c", subcore_axis_name="s",
    num_cores=_NC, num_subcores=_NS,
)


@functools.partial(
    pl.kernel,
    out_type=jax.ShapeDtypeStruct((_NW, _L), jnp.float32),
    mesh=_mesh,
    scratch_types=[
        pltpu.VMEM((_BPW,), jnp.int32),
        pltpu.VMEM((2, _CH, _DW), jnp.int32),
        pltpu.VMEM((2, _CH, _DW), jnp.int32),
        pltpu.VMEM((_L,), jnp.float32),
        pltpu.SemaphoreType.DMA,
        pltpu.SemaphoreType.DMA,
    ],
)
def _center_loss_sc(emb_hbm, lab_hbm, cen_hbm, out_hbm,
                    idx_v, e_v, c_v, acc_v, sem0, sem1):
    wid = lax.axis_index("s") * _NC + lax.axis_index("c")
    base = wid * _BPW
    pltpu.sync_copy(lab_hbm.at[pl.ds(base, _BPW)], idx_v)
    sems = (sem0, sem1)

    def start(ch, b):
        pltpu.async_copy(
            cen_hbm.at[idx_v.at[pl.ds(ch * _CH, _CH)]], c_v.at[b], sems[b])
        pltpu.async_copy(
            emb_hbm.at[pl.ds(base + ch * _CH, _CH), :], e_v.at[b], sems[b])

    def wait(b):
        pltpu.make_async_copy(emb_hbm.at[pl.ds(0, _CH), :], e_v.at[b],
                              sems[b]).wait()
        pltpu.make_async_copy(emb_hbm.at[pl.ds(0, _CH), :], c_v.at[b],
                              sems[b]).wait()

    hi_mask = jnp.full((_L,), jnp.int32(-65536))

    def expand(w):
        lo = lax.bitcast_convert_type(lax.shift_left(w, 16), jnp.float32)
        hi = lax.bitcast_convert_type(lax.bitwise_and(w, hi_mask), jnp.float32)
        return lo, hi

    def compute(b, a):
        def body(r, a):
            a0, a1, a2, a3 = a
            for j in range(_DW // (2 * _L)):
                ew0 = e_v[b, r, pl.ds((2 * j) * _L, _L)]
                cw0 = c_v[b, r, pl.ds((2 * j) * _L, _L)]
                ew1 = e_v[b, r, pl.ds((2 * j + 1) * _L, _L)]
                cw1 = c_v[b, r, pl.ds((2 * j + 1) * _L, _L)]
                elo0, ehi0 = expand(ew0)
                clo0, chi0 = expand(cw0)
                elo1, ehi1 = expand(ew1)
                clo1, chi1 = expand(cw1)
                d0 = elo0 - clo0
                d1 = ehi0 - chi0
                d2 = elo1 - clo1
                d3 = ehi1 - chi1
                a0 = a0 + d0 * d0
                a1 = a1 + d1 * d1
                a2 = a2 + d2 * d2
                a3 = a3 + d3 * d3
            return a0, a1, a2, a3
        return lax.fori_loop(0, _CH, body, a)

    start(0, 0)
    npairs = _NCHUNK // 2
    zero = jnp.zeros((_L,), jnp.float32)

    def pair_body(p, a):
        start(2 * p + 1, 1)
        wait(0)
        a = compute(0, a)

        @pl.when(p + 1 < npairs)
        def _():
            start(2 * p + 2, 0)

        wait(1)
        a = compute(1, a)
        return a

    a0, a1, a2, a3 = lax.fori_loop(0, npairs, pair_body,
                                   (zero, zero, zero, zero))

    acc_v[...] = (a0 + a1) + (a2 + a3)
    pltpu.sync_copy(acc_v, out_hbm.at[wid])


def _pack_bf16(x):
    u = jax.lax.bitcast_convert_type(x, jnp.uint32)
    r = (u + 0x7FFF + ((u >> 16) & 1)) >> 16
    h = x.shape[1] // 2
    w = r[:, :h] | (r[:, h:] << 16)
    return jax.lax.bitcast_convert_type(w, jnp.int32)


def kernel(embeddings, labels, centers):
    partials = _center_loss_sc(
        _pack_bf16(embeddings),
        labels.astype(jnp.int32),
        _pack_bf16(centers),
    )
    return _LAMBDA * (jnp.sum(partials) / jnp.float32(_B))


# speedup vs baseline: 4.4835x; 1.7660x over previous
import functools

import jax
import jax.numpy as jnp
from jax import lax
from jax.experimental import pallas as pl
from jax.experimental.pallas import tpu as pltpu
from jax.experimental.pallas import tpu_sc as plsc

_B = 16384
_D = 512
_DW = _D // 2
_NUM_CLASSES = 1000
_LAMBDA = 0.001

_NC, _NS, _L = 2, 16, 16
_NW = _NC * _NS
_BPW = _B // _NW
_CH = 64
_NCHUNK = _BPW // _CH

_mesh = plsc.VectorSubcoreMesh(
    core_axis_name="c", subcore_axis_name="s",
    num_cores=_NC, num_subcores=_NS,
)


@functools.partial(
    pl.kernel,
    out_type=jax.ShapeDtypeStruct((_NW, _L), jnp.float32),
    mesh=_mesh,
    scratch_types=[
        pltpu.VMEM((_BPW,), jnp.int32),
        pltpu.VMEM((2, _CH, _D), jnp.float32),
        pltpu.VMEM((2, _CH, _DW), jnp.int32),
        pltpu.VMEM((_L,), jnp.float32),
        pltpu.SemaphoreType.DMA,
        pltpu.SemaphoreType.DMA,
    ],
)
def _center_loss_sc(emb_hbm, lab_hbm, cen_hbm, out_hbm,
                    idx_v, e_v, c_v, acc_v, sem0, sem1):
    wid = lax.axis_index("s") * _NC + lax.axis_index("c")
    base = wid * _BPW
    pltpu.sync_copy(lab_hbm.at[pl.ds(base, _BPW)], idx_v)
    sems = (sem0, sem1)

    def start(ch, b):
        pltpu.async_copy(
            cen_hbm.at[idx_v.at[pl.ds(ch * _CH, _CH)]], c_v.at[b], sems[b])
        pltpu.async_copy(
            emb_hbm.at[pl.ds(base + ch * _CH, _CH), :], e_v.at[b], sems[b])

    def wait(b):
        pltpu.make_async_copy(emb_hbm.at[pl.ds(0, _CH), :], e_v.at[b],
                              sems[b]).wait()
        pltpu.make_async_copy(cen_hbm.at[pl.ds(0, _CH), :], c_v.at[b],
                              sems[b]).wait()

    hi_mask = jnp.full((_L,), jnp.int32(-65536))

    def expand(w):
        lo = lax.bitcast_convert_type(lax.shift_left(w, 16), jnp.float32)
        hi = lax.bitcast_convert_type(lax.bitwise_and(w, hi_mask), jnp.float32)
        return lo, hi

    def compute(b, a):
        def body(r, a):
            a0, a1, a2, a3 = a
            for j in range(_DW // (2 * _L)):
                cw0 = c_v[b, r, pl.ds((2 * j) * _L, _L)]
                cw1 = c_v[b, r, pl.ds((2 * j + 1) * _L, _L)]
                el0 = e_v[b, r, pl.ds((2 * j) * _L, _L)]
                el1 = e_v[b, r, pl.ds((2 * j + 1) * _L, _L)]
                eh0 = e_v[b, r, pl.ds(_DW + (2 * j) * _L, _L)]
                eh1 = e_v[b, r, pl.ds(_DW + (2 * j + 1) * _L, _L)]
                clo0, chi0 = expand(cw0)
                clo1, chi1 = expand(cw1)
                d0 = el0 - clo0
                d1 = eh0 - chi0
                d2 = el1 - clo1
                d3 = eh1 - chi1
                a0 = a0 + d0 * d0
                a1 = a1 + d1 * d1
                a2 = a2 + d2 * d2
                a3 = a3 + d3 * d3
            return a0, a1, a2, a3
        return lax.fori_loop(0, _CH, body, a)

    start(0, 0)
    npairs = _NCHUNK // 2
    zero = jnp.zeros((_L,), jnp.float32)

    def pair_body(p, a):
        start(2 * p + 1, 1)
        wait(0)
        a = compute(0, a)

        @pl.when(p + 1 < npairs)
        def _():
            start(2 * p + 2, 0)

        wait(1)
        a = compute(1, a)
        return a

    a0, a1, a2, a3 = lax.fori_loop(0, npairs, pair_body,
                                   (zero, zero, zero, zero))

    acc_v[...] = (a0 + a1) + (a2 + a3)
    pltpu.sync_copy(acc_v, out_hbm.at[wid])


def _pack_bf16(x):
    u = jax.lax.bitcast_convert_type(x, jnp.uint32)
    r = (u + 0x7FFF + ((u >> 16) & 1)) >> 16
    h = x.shape[1] // 2
    w = r[:, :h] | (r[:, h:] << 16)
    return jax.lax.bitcast_convert_type(w, jnp.int32)


def kernel(embeddings, labels, centers):
    partials = _center_loss_sc(
        embeddings,
        labels.astype(jnp.int32),
        _pack_bf16(centers),
    )
    return _LAMBDA * (jnp.sum(partials) / jnp.float32(_B))
